# Initial kernel scaffold; baseline (speedup 1.0000x reference)
import jax
import jax.numpy as jnp
from jax.experimental import pallas as pl


def kernel(x, edge_index, edge_mask, edge_attr, We1, be1, We2, be2, Wn1, bn1, Wn2, bn2):
    raise NotImplementedError("write your pallas kernel here")


# speedup vs baseline: 2.9013x; 2.9013x over previous
import functools

import jax
import jax.numpy as jnp
from jax import lax
from jax.experimental import pallas as pl
from jax.experimental.pallas import tpu as pltpu
from jax.experimental.pallas import tpu_sc as plsc

NC = 2
NS = 16
NW = NC * NS
L = 16

F32 = jnp.float32


def _tables_body(x_ref, wa_ref, wb_ref, be1_ref, xa_ref, xb_ref):
    x = x_ref[...]
    xa_ref[...] = jnp.dot(x, wa_ref[...], preferred_element_type=F32)
    xb_ref[...] = jnp.dot(x, wb_ref[...], preferred_element_type=F32) + be1_ref[...]


def _make_tables(n, d, h, bn):
    grid = (n // bn,)
    return pl.pallas_call(
        _tables_body,
        grid=grid,
        in_specs=[
            pl.BlockSpec((bn, d), lambda i: (i, 0)),
            pl.BlockSpec((d, h), lambda i: (0, 0)),
            pl.BlockSpec((d, h), lambda i: (0, 0)),
            pl.BlockSpec((1, h), lambda i: (0, 0)),
        ],
        out_specs=[
            pl.BlockSpec((bn, h), lambda i: (i, 0)),
            pl.BlockSpec((bn, h), lambda i: (i, 0)),
        ],
        out_shape=[
            jax.ShapeDtypeStruct((n, h), F32),
            jax.ShapeDtypeStruct((n, h), F32),
        ],
    )


def _gather_body(e, h, k, xa_hbm, xb_hbm, row_hbm, col_hbm, g_hbm,
                 idxa, idxb, ra, rb, sem):
    epw = e // NW
    wid = lax.axis_index("s") * NC + lax.axis_index("c")
    base = wid * epw

    def chunk(j, carry):
        off = base + j * k
        pltpu.sync_copy(row_hbm.at[pl.ds(off, k)], idxa)
        pltpu.sync_copy(col_hbm.at[pl.ds(off, k)], idxb)
        cp1 = pltpu.async_copy(xa_hbm.at[idxa], ra, sem)
        cp2 = pltpu.async_copy(xb_hbm.at[idxb], rb, sem)
        cp1.wait()
        cp2.wait()

        def addrow(i, c2):
            def addvec(v, c3):
                plsc.addupdate(rb.at[i, pl.ds(v * L, L)], ra[i, pl.ds(v * L, L)])
                return c3
            return lax.fori_loop(0, h // L, addvec, c2)

        lax.fori_loop(0, k, addrow, carry)
        pltpu.sync_copy(rb, g_hbm.at[pl.ds(off, k)])
        return carry

    lax.fori_loop(0, epw // k, chunk, 0)


def _make_gather(n, e, h, k):
    mesh = plsc.VectorSubcoreMesh(core_axis_name="c", subcore_axis_name="s")
    return pl.kernel(
        functools.partial(_gather_body, e, h, k),
        out_type=jax.ShapeDtypeStruct((e, h), F32),
        mesh=mesh,
        scratch_types=[
            pltpu.VMEM((k,), jnp.int32),
            pltpu.VMEM((k,), jnp.int32),
            pltpu.VMEM((k, h), F32),
            pltpu.VMEM((k, h), F32),
            pltpu.SemaphoreType.DMA,
        ],
    )


def _edge_body(g_ref, ea_ref, mask_ref, w1c_ref, w2_ref, be2_ref, ef_ref):
    pre = g_ref[...] + jnp.dot(ea_ref[...], w1c_ref[...], preferred_element_type=F32)
    hh = jnp.maximum(pre, 0.0)
    ef = jnp.dot(hh, w2_ref[...], preferred_element_type=F32) + be2_ref[...]
    ef_ref[...] = ef * mask_ref[...]


def _make_edge(e, de, h, be):
    grid = (e // be,)
    return pl.pallas_call(
        _edge_body,
        grid=grid,
        in_specs=[
            pl.BlockSpec((be, h), lambda i: (i, 0)),
            pl.BlockSpec((be, de), lambda i: (i, 0)),
            pl.BlockSpec((be, 1), lambda i: (i, 0)),
            pl.BlockSpec((de, h), lambda i: (0, 0)),
            pl.BlockSpec((h, h), lambda i: (0, 0)),
            pl.BlockSpec((1, h), lambda i: (0, 0)),
        ],
        out_specs=pl.BlockSpec((be, h), lambda i: (i, 0)),
        out_shape=jax.ShapeDtypeStruct((e, h), F32),
    )


def _scatter_body(n, e, h, k, ef_hbm, row_hbm, agg_hbm,
                  idx, rows, zbuf, agg_sh, sem):
    epw = e // NW
    ncx = (n + k - 1) // k
    cpt = (ncx + NS - 1) // NS
    cid = lax.axis_index("c")
    sid = lax.axis_index("s")
    wid = sid * NC + cid
    base = wid * epw

    def zrow(i, c2):
        def zvec(v, c3):
            zbuf[i, pl.ds(v * L, L)] = jnp.zeros((L,), F32)
            return c3
        return lax.fori_loop(0, h // L, zvec, c2)

    lax.fori_loop(0, k, zrow, 0)

    def zchunk(j, carry):
        ct = sid * cpt + j

        @pl.when(ct < ncx)
        def _():
            pltpu.sync_copy(zbuf, agg_sh.at[pl.ds(ct * k, k)])
        return carry

    lax.fori_loop(0, cpt, zchunk, 0)
    plsc.subcore_barrier()

    def chunk(j, carry):
        off = base + j * k
        pltpu.sync_copy(row_hbm.at[pl.ds(off, k)], idx)
        pltpu.async_copy(ef_hbm.at[pl.ds(off, k)], rows, sem).wait()
        pltpu.sync_copy(rows, agg_sh.at[idx], add=True)
        return carry

    lax.fori_loop(0, epw // k, chunk, 0)
    plsc.subcore_barrier()

    def dchunk(j, carry):
        ct = sid * cpt + j

        @pl.when(ct < ncx)
        def _():
            pltpu.sync_copy(agg_sh.at[pl.ds(ct * k, k)],
                            agg_hbm.at[cid, pl.ds(ct * k, k)])
        return carry

    lax.fori_loop(0, cpt, dchunk, 0)


def _make_scatter(n, e, h, k):
    mesh = plsc.VectorSubcoreMesh(core_axis_name="c", subcore_axis_name="s")
    return pl.kernel(
        functools.partial(_scatter_body, n, e, h, k),
        out_type=jax.ShapeDtypeStruct((NC, n, h), F32),
        mesh=mesh,
        scratch_types=[
            pltpu.VMEM((k,), jnp.int32),
            pltpu.VMEM((k, h), F32),
            pltpu.VMEM((k, h), F32),
            pltpu.VMEM_SHARED((n, h), F32),
            pltpu.SemaphoreType.DMA,
        ],
    )


def _node_body(x_ref, a0_ref, a1_ref, wn1a_ref, wn1b_ref, bn1_ref,
               wn2_ref, bn2_ref, out_ref):
    agg = a0_ref[...] + a1_ref[...]
    h2 = jnp.maximum(
        jnp.dot(x_ref[...], wn1a_ref[...], preferred_element_type=F32)
        + jnp.dot(agg, wn1b_ref[...], preferred_element_type=F32)
        + bn1_ref[...],
        0.0,
    )
    out_ref[...] = jnp.dot(h2, wn2_ref[...], preferred_element_type=F32) + bn2_ref[...]


def _make_node(n, d, h, bn):
    grid = (n // bn,)
    return pl.pallas_call(
        _node_body,
        grid=grid,
        in_specs=[
            pl.BlockSpec((bn, d), lambda i: (i, 0)),
            pl.BlockSpec((bn, h), lambda i: (i, 0)),
            pl.BlockSpec((bn, h), lambda i: (i, 0)),
            pl.BlockSpec((d, h), lambda i: (0, 0)),
            pl.BlockSpec((h, h), lambda i: (0, 0)),
            pl.BlockSpec((1, h), lambda i: (0, 0)),
            pl.BlockSpec((h, d), lambda i: (0, 0)),
            pl.BlockSpec((1, d), lambda i: (0, 0)),
        ],
        out_specs=pl.BlockSpec((bn, d), lambda i: (i, 0)),
        out_shape=jax.ShapeDtypeStruct((n, d), F32),
    )


def kernel(x, edge_index, edge_mask, edge_attr, We1, be1, We2, be2,
           Wn1, bn1, Wn2, bn2):
    n, d = x.shape
    e = edge_index.shape[1]
    de = edge_attr.shape[1]
    h = We2.shape[1]

    row = edge_index[0]
    col = edge_index[1]

    k = 80
    bn = 1000
    be = 2000

    xa, xb = _make_tables(n, d, h, bn)(
        x, We1[:d], We1[d:2 * d], be1.reshape(1, h))

    g = _make_gather(n, e, h, k)(xa, xb, row, col)

    ef = _make_edge(e, de, h, be)(
        g, edge_attr, edge_mask, We1[2 * d:], We2, be2.reshape(1, h))

    aggs = _make_scatter(n, e, h, k)(ef, row)

    x_out = _make_node(n, d, h, bn)(
        x, aggs[0], aggs[1], Wn1[:d], Wn1[d:], bn1.reshape(1, h),
        Wn2, bn2.reshape(1, d))

    return (x_out, ef)


# speedup vs baseline: 4.3283x; 1.4919x over previous
import functools

import jax
import jax.numpy as jnp
from jax import lax
from jax.experimental import pallas as pl
from jax.experimental.pallas import tpu as pltpu
from jax.experimental.pallas import tpu_sc as plsc

NC = 2
NS = 16
NW = NC * NS
L = 16

F32 = jnp.float32


def _tables_body(x_ref, wa_ref, wb_ref, be1_ref, xa_ref, xb_ref):
    x = x_ref[...]
    xa_ref[...] = jnp.dot(x, wa_ref[...], preferred_element_type=F32)
    xb_ref[...] = jnp.dot(x, wb_ref[...], preferred_element_type=F32) + be1_ref[...]


def _make_tables(n, d, h, bn):
    grid = (n // bn,)
    return pl.pallas_call(
        _tables_body,
        grid=grid,
        in_specs=[
            pl.BlockSpec((bn, d), lambda i: (i, 0)),
            pl.BlockSpec((d, h), lambda i: (0, 0)),
            pl.BlockSpec((d, h), lambda i: (0, 0)),
            pl.BlockSpec((1, h), lambda i: (0, 0)),
        ],
        out_specs=[
            pl.BlockSpec((bn, h), lambda i: (i, 0)),
            pl.BlockSpec((bn, h), lambda i: (i, 0)),
        ],
        out_shape=[
            jax.ShapeDtypeStruct((n, h), F32),
            jax.ShapeDtypeStruct((n, h), F32),
        ],
    )


def _gather_body(e, h, k, xa_hbm, xb_hbm, row_hbm, col_hbm, g_hbm,
                 idxa, idxb, ra, rb, sg0, sg1, sw0, sw1):
    epw = e // NW
    nch = epw // k
    wid = lax.axis_index("s") * NC + lax.axis_index("c")
    base = wid * epw
    sg = (sg0, sg1)
    sw = (sw0, sw1)

    pltpu.sync_copy(row_hbm.at[pl.ds(base, epw)], idxa)
    pltpu.sync_copy(col_hbm.at[pl.ds(base, epw)], idxb)

    def start(j, b):
        sl = pl.ds(j * k, k)
        pltpu.async_copy(xa_hbm.at[idxa.at[sl]], ra.at[b], sg[b])
        pltpu.async_copy(xb_hbm.at[idxb.at[sl]], rb.at[b], sg[b])

    def wait_gathers(b):
        d = pltpu.make_async_copy(xa_hbm.at[idxa.at[pl.ds(0, k)]], ra.at[b], sg[b])
        d.wait()
        d.wait()

    def wait_write(b):
        pltpu.make_async_copy(rb.at[b], g_hbm.at[pl.ds(base, k)], sw[b]).wait()

    start(0, 0)

    def outer(io, carry):
        for b in (0, 1):
            j = io * 2 + b

            @pl.when(j < nch)
            def _():
                @pl.when(j >= 1)
                def _():
                    wait_write(1 - b)

                @pl.when(j + 1 < nch)
                def _():
                    start(j + 1, 1 - b)

                wait_gathers(b)

                def addrow(i, c2):
                    def addvec(v, c3):
                        plsc.addupdate(rb.at[b, i, pl.ds(v * L, L)],
                                       ra[b, i, pl.ds(v * L, L)])
                        return c3
                    return lax.fori_loop(0, h // L, addvec, c2)

                lax.fori_loop(0, k, addrow, 0)
                pltpu.async_copy(rb.at[b], g_hbm.at[pl.ds(base + j * k, k)], sw[b])
        return carry

    lax.fori_loop(0, (nch + 1) // 2, outer, 0)
    wait_write((nch - 1) % 2)


def _make_gather(n, e, h, k):
    mesh = plsc.VectorSubcoreMesh(core_axis_name="c", subcore_axis_name="s")
    epw = e // NW
    return pl.kernel(
        functools.partial(_gather_body, e, h, k),
        out_type=jax.ShapeDtypeStruct((e, h), F32),
        mesh=mesh,
        scratch_types=[
            pltpu.VMEM((epw,), jnp.int32),
            pltpu.VMEM((epw,), jnp.int32),
            pltpu.VMEM((2, k, h), F32),
            pltpu.VMEM((2, k, h), F32),
            pltpu.SemaphoreType.DMA,
            pltpu.SemaphoreType.DMA,
            pltpu.SemaphoreType.DMA,
            pltpu.SemaphoreType.DMA,
        ],
    )


def _edge_body(g_ref, ea_ref, mask_ref, w1c_ref, w2_ref, be2_ref, ef_ref):
    pre = g_ref[...] + jnp.dot(ea_ref[...], w1c_ref[...], preferred_element_type=F32)
    hh = jnp.maximum(pre, 0.0)
    ef = jnp.dot(hh, w2_ref[...], preferred_element_type=F32) + be2_ref[...]
    ef_ref[...] = ef * mask_ref[...]


def _make_edge(e, de, h, be):
    grid = (e // be,)
    return pl.pallas_call(
        _edge_body,
        grid=grid,
        in_specs=[
            pl.BlockSpec((be, h), lambda i: (i, 0)),
            pl.BlockSpec((be, de), lambda i: (i, 0)),
            pl.BlockSpec((be, 1), lambda i: (i, 0)),
            pl.BlockSpec((de, h), lambda i: (0, 0)),
            pl.BlockSpec((h, h), lambda i: (0, 0)),
            pl.BlockSpec((1, h), lambda i: (0, 0)),
        ],
        out_specs=pl.BlockSpec((be, h), lambda i: (i, 0)),
        out_shape=jax.ShapeDtypeStruct((e, h), F32),
    )


def _scatter_body(n, e, h, k, ef_hbm, row_hbm, agg_hbm,
                  idx, rows, zbuf, agg_sh, sr0, sr1):
    epw = e // NW
    ncx = (n + k - 1) // k
    cpt = (ncx + NS - 1) // NS
    cid = lax.axis_index("c")
    sid = lax.axis_index("s")
    wid = sid * NC + cid
    base = wid * epw

    def zrow(i, c2):
        def zvec(v, c3):
            zbuf[i, pl.ds(v * L, L)] = jnp.zeros((L,), F32)
            return c3
        return lax.fori_loop(0, h // L, zvec, c2)

    lax.fori_loop(0, k, zrow, 0)

    def zchunk(j, carry):
        ct = sid * cpt + j

        @pl.when(ct < ncx)
        def _():
            pltpu.sync_copy(zbuf, agg_sh.at[pl.ds(ct * k, k)])
        return carry

    lax.fori_loop(0, cpt, zchunk, 0)
    plsc.subcore_barrier()

    sr = (sr0, sr1)

    def start(j, b):
        off = base + j * k
        pltpu.async_copy(row_hbm.at[pl.ds(off, k)], idx.at[b], sr[b])
        pltpu.async_copy(ef_hbm.at[pl.ds(off, k)], rows.at[b], sr[b])

    def wait_rows(b):
        pltpu.make_async_copy(row_hbm.at[pl.ds(base, k)], idx.at[b], sr[b]).wait()
        pltpu.make_async_copy(ef_hbm.at[pl.ds(base, k)], rows.at[b], sr[b]).wait()

    nch = epw // k
    start(0, 0)

    def chunk(io, carry):
        for b in (0, 1):
            j = io * 2 + b

            @pl.when(j < nch)
            def _():
                @pl.when(j + 1 < nch)
                def _():
                    start(j + 1, 1 - b)

                wait_rows(b)
                pltpu.sync_copy(rows.at[b], agg_sh.at[idx.at[b]], add=True)
        return carry

    lax.fori_loop(0, (nch + 1) // 2, chunk, 0)
    plsc.subcore_barrier()

    def dchunk(j, carry):
        ct = sid * cpt + j

        @pl.when(ct < ncx)
        def _():
            pltpu.sync_copy(agg_sh.at[pl.ds(ct * k, k)],
                            agg_hbm.at[cid, pl.ds(ct * k, k)])
        return carry

    lax.fori_loop(0, cpt, dchunk, 0)


def _make_scatter(n, e, h, k):
    mesh = plsc.VectorSubcoreMesh(core_axis_name="c", subcore_axis_name="s")
    epw = e // NW
    return pl.kernel(
        functools.partial(_scatter_body, n, e, h, k),
        out_type=jax.ShapeDtypeStruct((NC, n, h), F32),
        mesh=mesh,
        scratch_types=[
            pltpu.VMEM((2, k), jnp.int32),
            pltpu.VMEM((2, k, h), F32),
            pltpu.VMEM((k, h), F32),
            pltpu.VMEM_SHARED((n, h), F32),
            pltpu.SemaphoreType.DMA,
            pltpu.SemaphoreType.DMA,
        ],
    )


def _node_body(x_ref, a0_ref, a1_ref, wn1a_ref, wn1b_ref, bn1_ref,
               wn2_ref, bn2_ref, out_ref):
    agg = a0_ref[...] + a1_ref[...]
    h2 = jnp.maximum(
        jnp.dot(x_ref[...], wn1a_ref[...], preferred_element_type=F32)
        + jnp.dot(agg, wn1b_ref[...], preferred_element_type=F32)
        + bn1_ref[...],
        0.0,
    )
    out_ref[...] = jnp.dot(h2, wn2_ref[...], preferred_element_type=F32) + bn2_ref[...]


def _make_node(n, d, h, bn):
    grid = (n // bn,)
    return pl.pallas_call(
        _node_body,
        grid=grid,
        in_specs=[
            pl.BlockSpec((bn, d), lambda i: (i, 0)),
            pl.BlockSpec((bn, h), lambda i: (i, 0)),
            pl.BlockSpec((bn, h), lambda i: (i, 0)),
            pl.BlockSpec((d, h), lambda i: (0, 0)),
            pl.BlockSpec((h, h), lambda i: (0, 0)),
            pl.BlockSpec((1, h), lambda i: (0, 0)),
            pl.BlockSpec((h, d), lambda i: (0, 0)),
            pl.BlockSpec((1, d), lambda i: (0, 0)),
        ],
        out_specs=pl.BlockSpec((bn, d), lambda i: (i, 0)),
        out_shape=jax.ShapeDtypeStruct((n, d), F32),
    )


def kernel(x, edge_index, edge_mask, edge_attr, We1, be1, We2, be2,
           Wn1, bn1, Wn2, bn2):
    n, d = x.shape
    e = edge_index.shape[1]
    de = edge_attr.shape[1]
    h = We2.shape[1]

    row = edge_index[0]
    col = edge_index[1]

    k = 80
    bn = 1000
    be = 2000

    xa, xb = _make_tables(n, d, h, bn)(
        x, We1[:d], We1[d:2 * d], be1.reshape(1, h))

    g = _make_gather(n, e, h, k)(xa, xb, row, col)

    ef = _make_edge(e, de, h, be)(
        g, edge_attr, edge_mask, We1[2 * d:], We2, be2.reshape(1, h))

    aggs = _make_scatter(n, e, h, k)(ef, row)

    x_out = _make_node(n, d, h, bn)(
        x, aggs[0], aggs[1], Wn1[:d], Wn1[d:], bn1.reshape(1, h),
        Wn2, bn2.reshape(1, d))

    return (x_out, ef)
